# baseline (speedup 1.0000x reference)
import functools

import jax
import jax.numpy as jnp
from jax import lax
from jax.experimental import pallas as pl
from jax.experimental.pallas import tpu as pltpu
from jax.experimental.pallas import tpu_sc as plsc

VOCAB_TILE = 2048


def _sc_gather(table, idx):
    B = idx.shape[0]
    _, D = table.shape
    info = plsc.get_sparse_core_info()
    nw = info.num_cores * info.num_subcores
    b_per_w = B // nw
    mesh = plsc.VectorSubcoreMesh(core_axis_name="c", subcore_axis_name="s")

    @functools.partial(
        pl.kernel,
        mesh=mesh,
        out_type=jax.ShapeDtypeStruct((B, D), jnp.float32),
        scratch_types=[
            pltpu.VMEM((b_per_w,), jnp.int32),
            pltpu.VMEM((b_per_w, D), jnp.float32),
            pltpu.SemaphoreType.DMA,
        ],
        compiler_params=pltpu.CompilerParams(use_tc_tiling_on_sc=False),
    )
    def gather_kernel(table_hbm, idx_hbm, out_hbm, idx_v, rows_v, sem):
        wid = lax.axis_index("s") * info.num_cores + lax.axis_index("c")
        base = wid * b_per_w
        pltpu.sync_copy(idx_hbm.at[pl.ds(base, b_per_w)], idx_v)
        pltpu.async_copy(table_hbm.at[idx_v], rows_v, sem).wait()
        pltpu.sync_copy(rows_v, out_hbm.at[pl.ds(base, b_per_w)])

    return gather_kernel(table, idx)


def _lse_pass(embeds, W, b2d, V, nvt):
    B, D = embeds.shape

    def body(emb_ref, w_ref, b_ref, lse_ref, m_ref, s_ref):
        v = pl.program_id(0)
        logits = lax.dot_general(
            emb_ref[...], w_ref[...], (((1,), (1,)), ((), ())),
            preferred_element_type=jnp.float32) + b_ref[...]
        col = v * VOCAB_TILE + lax.broadcasted_iota(
            jnp.int32, (B, VOCAB_TILE), 1)
        logits = jnp.where(col < V, logits, -jnp.inf)
        tmax = jnp.max(logits, axis=1, keepdims=True)

        @pl.when(v == 0)
        def _():
            m_ref[...] = tmax
            s_ref[...] = jnp.sum(jnp.exp(logits - tmax), axis=1, keepdims=True)

        @pl.when(v > 0)
        def _():
            m_old = m_ref[...]
            m_new = jnp.maximum(m_old, tmax)
            s_ref[...] = (s_ref[...] * jnp.exp(m_old - m_new)
                          + jnp.sum(jnp.exp(logits - m_new), axis=1,
                                    keepdims=True))
            m_ref[...] = m_new

        @pl.when(v == nvt - 1)
        def _():
            lse_ref[...] = m_ref[...] + jnp.log(s_ref[...])

    return pl.pallas_call(
        body,
        grid=(nvt,),
        in_specs=[
            pl.BlockSpec((B, D), lambda v: (0, 0)),
            pl.BlockSpec((VOCAB_TILE, D), lambda v: (v, 0)),
            pl.BlockSpec((1, VOCAB_TILE), lambda v: (0, v)),
        ],
        out_specs=pl.BlockSpec((B, 1), lambda v: (0, 0)),
        out_shape=jax.ShapeDtypeStruct((B, 1), jnp.float32),
        scratch_shapes=[
            pltpu.VMEM((B, 1), jnp.float32),
            pltpu.VMEM((B, 1), jnp.float32),
        ],
    )(embeds, W, b2d)


def _write_pass(embeds, W, b2d, lse, V, nvt):
    B, D = embeds.shape

    def body(emb_ref, w_ref, b_ref, lse_ref, o_ref):
        logits = lax.dot_general(
            emb_ref[...], w_ref[...], (((1,), (1,)), ((), ())),
            preferred_element_type=jnp.float32) + b_ref[...]
        o_ref[...] = logits - lse_ref[...]

    return pl.pallas_call(
        body,
        grid=(nvt,),
        in_specs=[
            pl.BlockSpec((B, D), lambda v: (0, 0)),
            pl.BlockSpec((VOCAB_TILE, D), lambda v: (v, 0)),
            pl.BlockSpec((1, VOCAB_TILE), lambda v: (0, v)),
            pl.BlockSpec((B, 1), lambda v: (0, 0)),
        ],
        out_specs=pl.BlockSpec((B, VOCAB_TILE), lambda v: (0, v)),
        out_shape=jax.ShapeDtypeStruct((B, V), jnp.float32),
    )(embeds, W, b2d, lse)


def kernel(inputs, emb_table, W, b):
    V = W.shape[0]
    nvt = pl.cdiv(V, VOCAB_TILE)
    idx = inputs.astype(jnp.int32)
    embeds = _sc_gather(emb_table, idx)
    b2d = b.reshape(1, V)
    lse = _lse_pass(embeds, W, b2d, V, nvt)
    return _write_pass(embeds, W, b2d, lse, V, nvt)


# speedup vs baseline: 1.0059x; 1.0059x over previous
import functools

import jax
import jax.numpy as jnp
from jax import lax
from jax.experimental import pallas as pl
from jax.experimental.pallas import tpu as pltpu
from jax.experimental.pallas import tpu_sc as plsc

VOCAB_TILE = 2048


def _sc_gather(table, idx):
    B = idx.shape[0]
    _, D = table.shape
    info = plsc.get_sparse_core_info()
    nw = info.num_cores * info.num_subcores
    b_per_w = B // nw
    mesh = plsc.VectorSubcoreMesh(core_axis_name="c", subcore_axis_name="s")

    @functools.partial(
        pl.kernel,
        mesh=mesh,
        out_type=jax.ShapeDtypeStruct((B, D), jnp.float32),
        scratch_types=[
            pltpu.VMEM((b_per_w,), jnp.int32),
            pltpu.VMEM((b_per_w, D), jnp.float32),
            pltpu.SemaphoreType.DMA,
        ],
        compiler_params=pltpu.CompilerParams(use_tc_tiling_on_sc=False),
    )
    def gather_kernel(table_hbm, idx_hbm, out_hbm, idx_v, rows_v, sem):
        wid = lax.axis_index("s") * info.num_cores + lax.axis_index("c")
        base = wid * b_per_w
        pltpu.sync_copy(idx_hbm.at[pl.ds(base, b_per_w)], idx_v)
        pltpu.async_copy(table_hbm.at[idx_v], rows_v, sem).wait()
        pltpu.sync_copy(rows_v, out_hbm.at[pl.ds(base, b_per_w)])

    return gather_kernel(table, idx)


def _lse_pass(embeds, W, b2d, V, nvt):
    B, D = embeds.shape

    def body(emb_ref, w_ref, b_ref, lse_ref, m_ref, s_ref):
        v = pl.program_id(0)
        logits = lax.dot_general(
            emb_ref[...].astype(jnp.bfloat16), w_ref[...].astype(jnp.bfloat16),
            (((1,), (1,)), ((), ())),
            preferred_element_type=jnp.float32) + b_ref[...]
        col = v * VOCAB_TILE + lax.broadcasted_iota(
            jnp.int32, (B, VOCAB_TILE), 1)
        logits = jnp.where(col < V, logits, -jnp.inf)
        tmax = jnp.max(logits, axis=1, keepdims=True)

        @pl.when(v == 0)
        def _():
            m_ref[...] = tmax
            s_ref[...] = jnp.sum(jnp.exp(logits - tmax), axis=1, keepdims=True)

        @pl.when(v > 0)
        def _():
            m_old = m_ref[...]
            m_new = jnp.maximum(m_old, tmax)
            s_ref[...] = (s_ref[...] * jnp.exp(m_old - m_new)
                          + jnp.sum(jnp.exp(logits - m_new), axis=1,
                                    keepdims=True))
            m_ref[...] = m_new

        @pl.when(v == nvt - 1)
        def _():
            lse_ref[...] = m_ref[...] + jnp.log(s_ref[...])

    return pl.pallas_call(
        body,
        grid=(nvt,),
        in_specs=[
            pl.BlockSpec((B, D), lambda v: (0, 0)),
            pl.BlockSpec((VOCAB_TILE, D), lambda v: (v, 0)),
            pl.BlockSpec((1, VOCAB_TILE), lambda v: (0, v)),
        ],
        out_specs=pl.BlockSpec((B, 1), lambda v: (0, 0)),
        out_shape=jax.ShapeDtypeStruct((B, 1), jnp.float32),
        scratch_shapes=[
            pltpu.VMEM((B, 1), jnp.float32),
            pltpu.VMEM((B, 1), jnp.float32),
        ],
    )(embeds, W, b2d)


def _write_pass(embeds, W, b2d, lse, V, nvt):
    B, D = embeds.shape

    def body(emb_ref, w_ref, b_ref, lse_ref, o_ref):
        logits = lax.dot_general(
            emb_ref[...].astype(jnp.bfloat16), w_ref[...].astype(jnp.bfloat16),
            (((1,), (1,)), ((), ())),
            preferred_element_type=jnp.float32) + b_ref[...]
        o_ref[...] = logits - lse_ref[...]

    return pl.pallas_call(
        body,
        grid=(nvt,),
        in_specs=[
            pl.BlockSpec((B, D), lambda v: (0, 0)),
            pl.BlockSpec((VOCAB_TILE, D), lambda v: (v, 0)),
            pl.BlockSpec((1, VOCAB_TILE), lambda v: (0, v)),
            pl.BlockSpec((B, 1), lambda v: (0, 0)),
        ],
        out_specs=pl.BlockSpec((B, VOCAB_TILE), lambda v: (0, v)),
        out_shape=jax.ShapeDtypeStruct((B, V), jnp.float32),
    )(embeds, W, b2d, lse)


def kernel(inputs, emb_table, W, b):
    V = W.shape[0]
    nvt = pl.cdiv(V, VOCAB_TILE)
    idx = inputs.astype(jnp.int32)
    embeds = _sc_gather(emb_table, idx)
    b2d = b.reshape(1, V)
    lse = _lse_pass(embeds, W, b2d, V, nvt)
    return _write_pass(embeds, W, b2d, lse, V, nvt)
